# baseline (speedup 1.0000x reference)
import functools

import jax
import jax.numpy as jnp
from jax import lax
from jax.experimental import pallas as pl
from jax.experimental.pallas import tpu as pltpu
from jax.experimental.pallas import tpu_sc as plsc

_LN2 = 0.6931471805599453
_FMAX = 3.4028235e38


def _neg_log(v):
    bits = lax.bitcast_convert_type(v, jnp.int32)
    e = ((bits >> 23) & 0xFF) - 127
    mb = (bits & 0x007FFFFF) | 0x3F800000
    m = lax.bitcast_convert_type(mb, jnp.float32)
    big = m > 1.4142135
    m = jnp.where(big, m * 0.5, m)
    ef = e.astype(jnp.float32) + jnp.where(big, 1.0, 0.0)
    z = (m - 1.0) / (m + 1.0)
    z2 = z * z
    p = 1.0 + z2 * (1.0 / 3.0 + z2 * (0.2 + z2 * (1.0 / 7.0 + z2 * (1.0 / 9.0))))
    return -(ef * _LN2 + 2.0 * z * p)


def kernel(predictions, image_grids, target_boxes_grid):
    B, H, W = predictions.shape
    HW = H * W
    x3 = jnp.transpose(predictions, (1, 2, 0))
    grids_t = jnp.transpose(image_grids, (0, 3, 4, 1, 2))

    L = 16
    LG = 128
    NK = LG // L
    NPG = 4
    APG = 4
    G2T = (W // 8) * 8
    G2O = W - G2T
    G2N = (W + 1) - G2T

    mesh = plsc.VectorSubcoreMesh(core_axis_name="c", subcore_axis_name="s")

    @functools.partial(
        pl.kernel,
        mesh=mesh,
        out_type=jax.ShapeDtypeStruct((16, 128), jnp.float32),
        scratch_types=[
            pltpu.VMEM((APG, W, LG), jnp.float32),
            pltpu.VMEM((APG, W, G2N, LG), jnp.int32),
            pltpu.VMEM((8, LG), jnp.float32),
            pltpu.VMEM_SHARED((16, 8, LG), jnp.float32),
            pltpu.VMEM((NPG, 8, LG), jnp.float32),
            pltpu.VMEM_SHARED((8, 128), jnp.float32),
            pltpu.VMEM((8, 128), jnp.float32),
            pltpu.VMEM((L,), jnp.float32),
            pltpu.VMEM((8, 128), jnp.float32),
            pltpu.SemaphoreType.DMA,
            pltpu.SemaphoreType.DMA,
        ],
    )
    def sck(x_hbm, g_hbm, out_hbm, xv, gv, part, shared, red, shared2, red2,
            outv, outb, sem1, sem2):
        cid = lax.axis_index("c")
        sid = lax.axis_index("s")
        pg = sid // NPG
        lg = sid % NPG
        lane0 = pl.multiple_of((cid * NPG + lg) * LG, LG)
        a0 = pg * APG
        nl = H - (NPG - 1) * APG
        na = jnp.where(pg < NPG - 1, APG, nl)

        @pl.when(pg < NPG - 1)
        def _():
            cpx = pltpu.make_async_copy(
                x_hbm.at[pl.ds(a0, APG), :, pl.ds(lane0, LG)], xv, sem1)
            cpg = pltpu.make_async_copy(
                g_hbm.at[H, pl.ds(a0, APG), :, pl.ds(G2T, G2N), pl.ds(lane0, LG)],
                gv, sem2)
            cpx.start()
            cpg.start()
            cpx.wait()
            cpg.wait()

        @pl.when(pg == NPG - 1)
        def _():
            src = pl.ds((NPG - 1) * APG, nl)
            cpx = pltpu.make_async_copy(
                x_hbm.at[src, :, pl.ds(lane0, LG)], xv.at[pl.ds(0, nl)], sem1)
            cpg = pltpu.make_async_copy(
                g_hbm.at[H, src, :, pl.ds(G2T, G2N), pl.ds(lane0, LG)],
                gv.at[pl.ds(0, nl)], sem2)
            cpx.start()
            cpg.start()
            cpx.wait()
            cpg.wait()

        zeros = jnp.zeros((L,), jnp.float32)

        def body(a, carry):
            nxt = []
            for k in range(NK):
                s_pm, cnt, s_p = carry[k]
                for b in range(W):
                    x = xv[a, b, pl.ds(k * L, L)]
                    mm = gv[a, b, G2O, pl.ds(k * L, L)].astype(jnp.float32)
                    s_pm = s_pm + x * mm
                    cnt = cnt + mm
                    s_p = s_p + x
                nxt.append((s_pm, cnt, s_p))
            return tuple(nxt)

        accs = lax.fori_loop(
            0, na, body, tuple((zeros, zeros, zeros) for _ in range(NK))
        )
        for k in range(NK):
            part[0, pl.ds(k * L, L)] = accs[k][0]
            part[1, pl.ds(k * L, L)] = accs[k][1]
            part[2, pl.ds(k * L, L)] = accs[k][2]
        pltpu.sync_copy(part, shared.at[sid])
        plsc.subcore_barrier()

        @pl.when(sid < NPG)
        def _():
            for kk in range(NPG):
                pltpu.sync_copy(shared.at[kk * NPG + sid], red.at[kk])
            contrib = zeros
            for k in range(NK):
                s_pm = zeros
                cnt = zeros
                s_p = zeros
                for kk in range(NPG):
                    s_pm = s_pm + red[kk, 0, pl.ds(k * L, L)]
                    cnt = cnt + red[kk, 1, pl.ds(k * L, L)]
                    s_p = s_p + red[kk, 2, pl.ds(k * L, L)]
                mean_t = s_pm / cnt
                lt = jnp.where(mean_t > 0.0, _neg_log(mean_t), jnp.inf)
                mean_b = (s_p - s_pm) / (float(HW) - cnt)
                arg = 1.0 - mean_b
                lb = jnp.where(
                    arg > 0.0, _neg_log(arg), jnp.where(arg == 0.0, _FMAX, 0.0)
                )
                contrib = contrib + lt + lb
            outv[...] = contrib * (1.0 / B)
            pltpu.sync_copy(outv, shared2.at[sid, pl.ds(0, L)])

        plsc.subcore_barrier()

        @pl.when(sid == 0)
        def _():
            pltpu.sync_copy(shared2, red2)
            tot = (
                red2[0, pl.ds(0, L)]
                + red2[1, pl.ds(0, L)]
                + red2[2, pl.ds(0, L)]
                + red2[3, pl.ds(0, L)]
            )
            n = L
            while n > 1:
                n //= 2
                idx = (lax.iota(jnp.int32, L) + n) % L
                rot = lax.gather(
                    tot,
                    idx[:, None],
                    lax.GatherDimensionNumbers(
                        offset_dims=(),
                        collapsed_slice_dims=(0,),
                        start_index_map=(0,),
                    ),
                    slice_sizes=(1,),
                    mode=lax.GatherScatterMode.PROMISE_IN_BOUNDS,
                )
                tot = tot + rot
            outb[0, pl.ds(0, L)] = tot
            row0 = pl.multiple_of(cid * 8, 8)
            pltpu.sync_copy(outb, out_hbm.at[pl.ds(row0, 8), :])

    out = sck(x3, grids_t)
    return out[0, 0] + out[8, 0]


# speedup vs baseline: 1.0722x; 1.0722x over previous
import functools

import jax
import jax.numpy as jnp
from jax import lax
from jax.experimental import pallas as pl
from jax.experimental.pallas import tpu as pltpu
from jax.experimental.pallas import tpu_sc as plsc

_LN2 = 0.6931471805599453
_FMAX = 3.4028235e38


def _neg_log(v):
    bits = lax.bitcast_convert_type(v, jnp.int32)
    e = ((bits >> 23) & 0xFF) - 127
    mb = (bits & 0x007FFFFF) | 0x3F800000
    m = lax.bitcast_convert_type(mb, jnp.float32)
    big = m > 1.4142135
    m = jnp.where(big, m * 0.5, m)
    ef = e.astype(jnp.float32) + jnp.where(big, 1.0, 0.0)
    z = (m - 1.0) / (m + 1.0)
    z2 = z * z
    p = 1.0 + z2 * (1.0 / 3.0 + z2 * (0.2 + z2 * (1.0 / 7.0 + z2 * (1.0 / 9.0))))
    return -(ef * _LN2 + 2.0 * z * p)


def kernel(predictions, image_grids, target_boxes_grid):
    B, H, W = predictions.shape
    HW = H * W
    x3 = jnp.transpose(predictions, (1, 2, 0))
    grids_t = jnp.transpose(image_grids, (0, 3, 4, 1, 2))

    L = 16
    LG = 128
    NK = LG // L
    NPG = 4
    QPG = HW // NPG
    APG = 4
    G2T = (W // 8) * 8
    G2O = W - G2T
    G2N = (W + 1) - G2T

    mesh = plsc.VectorSubcoreMesh(core_axis_name="c", subcore_axis_name="s")

    @functools.partial(
        pl.kernel,
        mesh=mesh,
        out_type=jax.ShapeDtypeStruct((16, 128), jnp.float32),
        scratch_types=[
            pltpu.VMEM((APG, W, LG), jnp.float32),
            pltpu.VMEM((APG, W, G2N, LG), jnp.int32),
            pltpu.VMEM((8, LG), jnp.float32),
            pltpu.VMEM_SHARED((16, 8, LG), jnp.float32),
            pltpu.VMEM((NPG, 8, LG), jnp.float32),
            pltpu.VMEM_SHARED((8, 128), jnp.float32),
            pltpu.VMEM((8, 128), jnp.float32),
            pltpu.VMEM((L,), jnp.float32),
            pltpu.VMEM((8, 128), jnp.float32),
            pltpu.SemaphoreType.DMA,
            pltpu.SemaphoreType.DMA,
        ],
    )
    def sck(x_hbm, g_hbm, out_hbm, xv, gv, part, shared, red, shared2, red2,
            outv, outb, sem1, sem2):
        cid = lax.axis_index("c")
        sid = lax.axis_index("s")
        pg = sid // NPG
        lg = sid % NPG
        lane0 = pl.multiple_of((cid * NPG + lg) * LG, LG)
        q0 = pg * QPG
        a_lo = q0 // W

        cpx = pltpu.make_async_copy(
            x_hbm.at[pl.ds(a_lo, APG), :, pl.ds(lane0, LG)], xv, sem1)
        cpg = pltpu.make_async_copy(
            g_hbm.at[H, pl.ds(a_lo, APG), :, pl.ds(G2T, G2N), pl.ds(lane0, LG)],
            gv, sem2)
        cpx.start()
        cpg.start()
        cpx.wait()
        cpg.wait()

        zeros = jnp.zeros((L,), jnp.float32)

        def body(j, carry):
            q = q0 + j
            a = q // W - a_lo
            b = q % W
            nxt = []
            for k in range(NK):
                s_pm, cnt, s_p = carry[k]
                x = xv[a, b, pl.ds(k * L, L)]
                mm = gv[a, b, G2O, pl.ds(k * L, L)].astype(jnp.float32)
                nxt.append((s_pm + x * mm, cnt + mm, s_p + x))
            return tuple(nxt)

        accs = lax.fori_loop(
            0, QPG, body, tuple((zeros, zeros, zeros) for _ in range(NK))
        )
        for k in range(NK):
            part[0, pl.ds(k * L, L)] = accs[k][0]
            part[1, pl.ds(k * L, L)] = accs[k][1]
            part[2, pl.ds(k * L, L)] = accs[k][2]
        pltpu.sync_copy(part, shared.at[sid])
        plsc.subcore_barrier()

        @pl.when(sid < NPG)
        def _():
            for kk in range(NPG):
                pltpu.sync_copy(shared.at[kk * NPG + sid], red.at[kk])

            def cbody(k, contrib):
                o = pl.multiple_of(k * L, L)
                s_pm = zeros
                cnt = zeros
                s_p = zeros
                for kk in range(NPG):
                    s_pm = s_pm + red[kk, 0, pl.ds(o, L)]
                    cnt = cnt + red[kk, 1, pl.ds(o, L)]
                    s_p = s_p + red[kk, 2, pl.ds(o, L)]
                mean_t = s_pm / cnt
                lt = jnp.where(mean_t > 0.0, _neg_log(mean_t), jnp.inf)
                mean_b = (s_p - s_pm) / (float(HW) - cnt)
                arg = 1.0 - mean_b
                lb = jnp.where(
                    arg > 0.0, _neg_log(arg), jnp.where(arg == 0.0, _FMAX, 0.0)
                )
                return contrib + lt + lb

            contrib = lax.fori_loop(0, NK, cbody, zeros)
            outv[...] = contrib * (1.0 / B)
            pltpu.sync_copy(outv, shared2.at[sid, pl.ds(0, L)])

        plsc.subcore_barrier()

        @pl.when(sid == 0)
        def _():
            pltpu.sync_copy(shared2, red2)
            tot = (
                red2[0, pl.ds(0, L)]
                + red2[1, pl.ds(0, L)]
                + red2[2, pl.ds(0, L)]
                + red2[3, pl.ds(0, L)]
            )
            n = L
            while n > 1:
                n //= 2
                idx = (lax.iota(jnp.int32, L) + n) % L
                rot = lax.gather(
                    tot,
                    idx[:, None],
                    lax.GatherDimensionNumbers(
                        offset_dims=(),
                        collapsed_slice_dims=(0,),
                        start_index_map=(0,),
                    ),
                    slice_sizes=(1,),
                    mode=lax.GatherScatterMode.PROMISE_IN_BOUNDS,
                )
                tot = tot + rot
            outb[0, pl.ds(0, L)] = tot
            row0 = pl.multiple_of(cid * 8, 8)
            pltpu.sync_copy(outb, out_hbm.at[pl.ds(row0, 8), :])

    out = sck(x3, grids_t)
    return out[0, 0] + out[8, 0]
